# baseline (device time: 62819 ns/iter reference)
import jax
import jax.numpy as jnp
from jax import lax
from jax.experimental import pallas as pl
from jax.experimental.pallas import tpu as pltpu

N_DEV = 4
N_PHASES = 1 + 2 * 3


def kernel(x, Win0, Wout0, Win1, Wout1, Win2, Wout2):
    m_per, d = x.shape
    m = N_DEV * m_per
    dh = d // 2

    def body(x_ref, win0_ref, wout0_ref, win1_ref, wout1_ref, win2_ref,
             wout2_ref, out_ref, xfull, xmid, rs_buf, p0_buf, own_buf,
             send_sems, recv_sems):
        me = lax.axis_index("i")

        def rows_of(pos):
            return pl.ds((pos % N_DEV) * m_per, m_per)

        barrier_sem = pltpu.get_barrier_semaphore()
        for o in range(1, N_DEV):
            pl.semaphore_signal(
                barrier_sem, inc=1,
                device_id=((me + o) % N_DEV,),
                device_id_type=pl.DeviceIdType.MESH,
            )
        pl.semaphore_wait(barrier_sem, N_DEV - 1)

        def ag_sends(phase, buf):
            rdmas = []
            for h in range(2):
                for o in range(1, N_DEV):
                    sl = (o - 1) * 2 + h
                    rdma = pltpu.make_async_remote_copy(
                        src_ref=buf.at[rows_of(me), pl.ds(h * dh, dh)],
                        dst_ref=buf.at[rows_of(me), pl.ds(h * dh, dh)],
                        send_sem=send_sems.at[phase, sl],
                        recv_sem=recv_sems.at[phase, sl],
                        device_id=((me + o) % N_DEV,),
                        device_id_type=pl.DeviceIdType.MESH,
                    )
                    rdma.start()
                    rdmas.append(rdma)
            return rdmas

        def recv_wait(phase, sl, dst):
            pltpu.make_async_remote_copy(
                src_ref=dst, dst_ref=dst,
                send_sem=send_sems.at[phase, sl],
                recv_sem=recv_sems.at[phase, sl],
                device_id=(me,),
                device_id_type=pl.DeviceIdType.MESH,
            ).wait_recv()

        def matmul(a, b):
            return jnp.dot(a.astype(jnp.bfloat16), b.astype(jnp.bfloat16),
                           preferred_element_type=jnp.float32)

        xfull[rows_of(me), :] = x_ref[:, :].astype(jnp.bfloat16)
        prev_ag = ag_sends(0, xfull)
        prev_ag_phase = 0

        layers = [
            (win0_ref, wout0_ref, xfull, xmid),
            (win1_ref, wout1_ref, xmid, xfull),
            (win2_ref, wout2_ref, xfull, xmid),
        ]
        for l, (win, wout, xin, dst) in enumerate(layers):
            rs_phase, ag_phase = 1 + 2 * l, 2 + 2 * l

            own_hid = jnp.maximum(matmul(xin[rows_of(me), :], win[:, :]), 0.0)
            own_buf[:, :] = matmul(own_hid, wout[:, :])

            for o in range(1, N_DEV):
                recv_wait(prev_ag_phase, (N_DEV - o - 1) * 2,
                          xin.at[rows_of(me + o), pl.ds(0, dh)])
                p0_buf[o - 1] = matmul(xin[rows_of(me + o), pl.ds(0, dh)],
                                       win[pl.ds(0, dh), :])
            rs_rdmas = []
            for o in range(1, N_DEV):
                recv_wait(prev_ag_phase, (N_DEV - o - 1) * 2 + 1,
                          xin.at[rows_of(me + o), pl.ds(dh, dh)])
                hid = jnp.maximum(
                    p0_buf[o - 1] + matmul(xin[rows_of(me + o), pl.ds(dh, dh)],
                                           win[pl.ds(dh, dh), :]),
                    0.0,
                )
                xin[rows_of(me + o), :] = matmul(hid, wout[:, :]).astype(
                    jnp.bfloat16)
                rdma = pltpu.make_async_remote_copy(
                    src_ref=xin.at[rows_of(me + o), :],
                    dst_ref=rs_buf.at[l, o - 1],
                    send_sem=send_sems.at[rs_phase, o - 1],
                    recv_sem=recv_sems.at[rs_phase, o - 1],
                    device_id=((me + o) % N_DEV,),
                    device_id_type=pl.DeviceIdType.MESH,
                )
                rdma.start()
                rs_rdmas.append(rdma)

            for r in prev_ag:
                r.wait_send()
            for r in rs_rdmas:
                r.wait_send()

            red = own_buf[:, :]
            for s in range(N_DEV - 1):
                recv_wait(rs_phase, s, rs_buf.at[l, s])
                red = red + rs_buf[l, s].astype(jnp.float32)
            dst[rows_of(me), :] = red.astype(jnp.bfloat16)
            prev_ag = ag_sends(ag_phase, dst)
            prev_ag_phase = ag_phase

        for o in range(1, N_DEV):
            for h in range(2):
                recv_wait(prev_ag_phase, (N_DEV - o - 1) * 2 + h,
                          xmid.at[rows_of(me + o), pl.ds(h * dh, dh)])
        out_ref[:, :] = xmid[:, :].astype(jnp.float32)
        for r in prev_ag:
            r.wait_send()

    return pl.pallas_call(
        body,
        out_shape=jax.ShapeDtypeStruct((m, d), jnp.float32),
        in_specs=[pl.BlockSpec(memory_space=pltpu.VMEM)] * 7,
        out_specs=pl.BlockSpec(memory_space=pltpu.VMEM),
        scratch_shapes=[
            pltpu.VMEM((m, d), jnp.bfloat16),
            pltpu.VMEM((m, d), jnp.bfloat16),
            pltpu.VMEM((3, N_DEV - 1, m_per, d), jnp.bfloat16),
            pltpu.VMEM((N_DEV - 1, m_per, Win0.shape[1]), jnp.float32),
            pltpu.VMEM((m_per, d), jnp.float32),
            pltpu.SemaphoreType.DMA((N_PHASES, 2 * (N_DEV - 1))),
            pltpu.SemaphoreType.DMA((N_PHASES, 2 * (N_DEV - 1))),
        ],
        compiler_params=pltpu.CompilerParams(
            collective_id=0, vmem_limit_bytes=100 * 1024 * 1024
        ),
    )(x, Win0, Wout0, Win1, Wout1, Win2, Wout2)


# device time: 62574 ns/iter; 1.0039x vs baseline; 1.0039x over previous
import jax
import jax.numpy as jnp
from jax import lax
from jax.experimental import pallas as pl
from jax.experimental.pallas import tpu as pltpu

N_DEV = 4
N_PHASES = 1 + 2 * 3


def kernel(x, Win0, Wout0, Win1, Wout1, Win2, Wout2):
    m_per, d = x.shape
    m = N_DEV * m_per
    dh = d // 2

    def body(x_ref, win0_ref, wout0_ref, win1_ref, wout1_ref, win2_ref,
             wout2_ref, out_ref, xfull, xmid, rs_buf, p0_buf, own_buf,
             send_sems, recv_sems):
        me = lax.axis_index("i")

        def rows_of(pos):
            return pl.ds((pos % N_DEV) * m_per, m_per)

        barrier_sem = pltpu.get_barrier_semaphore()
        for o in range(1, N_DEV):
            pl.semaphore_signal(
                barrier_sem, inc=1,
                device_id=((me + o) % N_DEV,),
                device_id_type=pl.DeviceIdType.MESH,
            )
        pl.semaphore_wait(barrier_sem, N_DEV - 1)

        def ag_sends(phase, buf):
            rdmas = []
            for h in range(2):
                for o in range(1, N_DEV):
                    sl = (o - 1) * 2 + h
                    rdma = pltpu.make_async_remote_copy(
                        src_ref=buf.at[rows_of(me), pl.ds(h * dh, dh)],
                        dst_ref=buf.at[rows_of(me), pl.ds(h * dh, dh)],
                        send_sem=send_sems.at[phase, sl],
                        recv_sem=recv_sems.at[phase, sl],
                        device_id=((me + o) % N_DEV,),
                        device_id_type=pl.DeviceIdType.MESH,
                    )
                    rdma.start()
                    rdmas.append(rdma)
            return rdmas

        def recv_wait(phase, sl, dst):
            pltpu.make_async_remote_copy(
                src_ref=dst, dst_ref=dst,
                send_sem=send_sems.at[phase, sl],
                recv_sem=recv_sems.at[phase, sl],
                device_id=(me,),
                device_id_type=pl.DeviceIdType.MESH,
            ).wait_recv()

        def matmul(a, b):
            return jnp.dot(a, b, preferred_element_type=jnp.float32)

        xfull[rows_of(me), :] = x_ref[:, :].astype(jnp.bfloat16)
        prev_ag = ag_sends(0, xfull)
        prev_ag_phase = 0

        layers = [
            (win0_ref, wout0_ref, xfull, xmid),
            (win1_ref, wout1_ref, xmid, xfull),
            (win2_ref, wout2_ref, xfull, xmid),
        ]
        for l, (win, wout, xin, dst) in enumerate(layers):
            rs_phase, ag_phase = 1 + 2 * l, 2 + 2 * l

            own_hid = jnp.maximum(matmul(xin[rows_of(me), :], win[:, :]), 0.0)
            own_buf[:, :] = matmul(own_hid, wout[:, :])

            for o in range(1, N_DEV):
                recv_wait(prev_ag_phase, (N_DEV - o - 1) * 2,
                          xin.at[rows_of(me + o), pl.ds(0, dh)])
                p0_buf[o - 1] = matmul(xin[rows_of(me + o), pl.ds(0, dh)],
                                       win[pl.ds(0, dh), :])
            rs_rdmas = []
            for o in range(1, N_DEV):
                recv_wait(prev_ag_phase, (N_DEV - o - 1) * 2 + 1,
                          xin.at[rows_of(me + o), pl.ds(dh, dh)])
                hid = jnp.maximum(
                    p0_buf[o - 1] + matmul(xin[rows_of(me + o), pl.ds(dh, dh)],
                                           win[pl.ds(dh, dh), :]),
                    0.0,
                )
                xin[rows_of(me + o), :] = matmul(hid, wout[:, :]).astype(
                    jnp.bfloat16)
                rdma = pltpu.make_async_remote_copy(
                    src_ref=xin.at[rows_of(me + o), :],
                    dst_ref=rs_buf.at[l, o - 1],
                    send_sem=send_sems.at[rs_phase, o - 1],
                    recv_sem=recv_sems.at[rs_phase, o - 1],
                    device_id=((me + o) % N_DEV,),
                    device_id_type=pl.DeviceIdType.MESH,
                )
                rdma.start()
                rs_rdmas.append(rdma)

            for r in prev_ag:
                r.wait_send()
            for r in rs_rdmas:
                r.wait_send()

            red = own_buf[:, :]
            for s in range(N_DEV - 1):
                recv_wait(rs_phase, s, rs_buf.at[l, s])
                red = red + rs_buf[l, s].astype(jnp.float32)
            dst[rows_of(me), :] = red.astype(jnp.bfloat16)
            prev_ag = ag_sends(ag_phase, dst)
            prev_ag_phase = ag_phase

        for o in range(1, N_DEV):
            for h in range(2):
                recv_wait(prev_ag_phase, (N_DEV - o - 1) * 2 + h,
                          xmid.at[rows_of(me + o), pl.ds(h * dh, dh)])
        out_ref[:, :] = xmid[:, :].astype(jnp.float32)
        for r in prev_ag:
            r.wait_send()

    return pl.pallas_call(
        body,
        out_shape=jax.ShapeDtypeStruct((m, d), jnp.float32),
        in_specs=[pl.BlockSpec(memory_space=pltpu.VMEM)] * 7,
        out_specs=pl.BlockSpec(memory_space=pltpu.VMEM),
        scratch_shapes=[
            pltpu.VMEM((m, d), jnp.bfloat16),
            pltpu.VMEM((m, d), jnp.bfloat16),
            pltpu.VMEM((3, N_DEV - 1, m_per, d), jnp.bfloat16),
            pltpu.VMEM((N_DEV - 1, m_per, Win0.shape[1]), jnp.float32),
            pltpu.VMEM((m_per, d), jnp.float32),
            pltpu.SemaphoreType.DMA((N_PHASES, 2 * (N_DEV - 1))),
            pltpu.SemaphoreType.DMA((N_PHASES, 2 * (N_DEV - 1))),
        ],
        compiler_params=pltpu.CompilerParams(
            collective_id=0, vmem_limit_bytes=100 * 1024 * 1024
        ),
    )(x, Win0, Wout0, Win1, Wout1, Win2, Wout2)
